# Initial kernel scaffold; baseline (speedup 1.0000x reference)
import jax
import jax.numpy as jnp
from jax.experimental import pallas as pl


def kernel(positions, cell, lengths, partial_charges, atomic_energies, edge_index, batch, n_atoms):
    raise NotImplementedError("write your pallas kernel here")


# speedup vs baseline: 15.1754x; 15.1754x over previous
import functools

import numpy as np
import jax
import jax.numpy as jnp
from jax import lax
from jax.experimental import pallas as pl
from jax.experimental.pallas import tpu as pltpu
from jax.experimental.pallas import tpu_sc as plsc

KE = 14.399645351950548
KE_HALF = KE / 2.0
ALPHA = 1.0
R_CUTOFF = 5.0
K_MAX = 10
TWO_PI = 2.0 * np.pi
SQRT_PI = float(np.sqrt(np.pi))
SELF_COEF = KE * ALPHA / SQRT_PI

AS_P = 0.3275911
AS_A1 = 0.254829592
AS_A2 = -0.284496736
AS_A3 = 1.421413741
AS_A4 = -1.453152027
AS_A5 = 1.061405429

INV_TWO_PI = 0.15915494309189535
TP_HI = 6.283203125
TP_MID = -1.7818063497543335e-05
TP_LO = 2.430837753308879e-10

N_ATOMS = 10000
N_PAD = 10240
BN = 512
NB = N_PAD // BN
B_STRUCT = 16
E_EDGES = 640000
NW = 32
EW_ROWS = 157
EW = EW_ROWS * 128
E_PAD = NW * EW
ACC_PAD = N_PAD


def _kmesh_half():
    k_idxs = np.arange(-K_MAX, K_MAX + 1).astype(np.float32)
    ku, kv, kw = np.meshgrid(k_idxs[K_MAX:], k_idxs, k_idxs, indexing='ij')
    km = np.stack([ku, kv, kw], axis=-1).reshape(-1, 3)
    ksq = (km ** 2).sum(-1)
    mask = (ksq > 0) & (ksq <= K_MAX ** 2)
    km = km[mask]
    sym = np.where(km[:, 0] == 0.0, 1.0, 2.0).astype(np.float32)
    return km.astype(np.float32), sym


_KM, _SYM = _kmesh_half()
K_REAL = _KM.shape[0]
K_PAD = ((K_REAL + 127) // 128) * 128
_KM_T = np.zeros((3, K_PAD), np.float32)
_KM_T[:, :K_REAL] = _KM.T
_SYM_PAD = np.zeros((1, K_PAD), np.float32)
_SYM_PAD[0, :K_REAL] = _SYM


def _sc_real_body(idxi_hbm, idxj_hbm, len_hbm, q_hbm, out_hbm,
                  idxi_v, idxj_v, len_v, vals_v, q_v, z_v, acc_sh):
    c = lax.axis_index("c")
    s = lax.axis_index("s")
    wid = c * 16 + s

    def zbody(i, carry):
        z_v[pl.ds(i * 16, 16)] = jnp.zeros((16,), jnp.float32)
        return carry
    lax.fori_loop(0, (ACC_PAD // 16) // 16, zbody, 0)
    pltpu.sync_copy(z_v, acc_sh.at[pl.ds(s * (ACC_PAD // 16), ACC_PAD // 16)])

    pltpu.sync_copy(idxi_hbm.at[wid], idxi_v)
    pltpu.sync_copy(idxj_hbm.at[wid], idxj_v)
    pltpu.sync_copy(len_hbm.at[wid], len_v)
    pltpu.sync_copy(q_hbm, q_v)

    def cbody(t, carry):
        j = t // 8
        col = (t % 8) * 16
        ii = idxi_v[j, pl.ds(col, 16)]
        jj = idxj_v[j, pl.ds(col, 16)]
        ll = len_v[j, pl.ds(col, 16)]
        qi = plsc.load_gather(q_v, [ii])
        qj = plsc.load_gather(q_v, [jj])
        tt = 1.0 / (1.0 + AS_P * ll)
        poly = tt * (AS_A1 + tt * (AS_A2 + tt * (AS_A3 + tt * (AS_A4 + tt * AS_A5))))
        erfc_v = poly * jnp.exp(-ll * ll)
        pv = KE_HALF * qi * qj * erfc_v / ll
        pv = jnp.where(ll <= R_CUTOFF, pv, jnp.zeros((16,), jnp.float32))
        vals_v[j, pl.ds(col, 16)] = pv
        return carry
    lax.fori_loop(0, EW_ROWS * 8, cbody, 0)

    plsc.subcore_barrier()

    def sbody(j, carry):
        pltpu.sync_copy(vals_v.at[j], acc_sh.at[idxi_v.at[j]], add=True)
        return carry
    lax.fori_loop(0, EW_ROWS, sbody, 0)

    plsc.subcore_barrier()

    @pl.when(s == 0)
    def _():
        pltpu.sync_copy(acc_sh, out_hbm.at[c])


def _real_space_sc(idxi3, idxj3, len3, q_pad):
    mesh = plsc.VectorSubcoreMesh(core_axis_name="c", subcore_axis_name="s")
    kfn = pl.kernel(
        _sc_real_body,
        mesh=mesh,
        compiler_params=pltpu.CompilerParams(needs_layout_passes=False),
        out_type=jax.ShapeDtypeStruct((2, ACC_PAD), jnp.float32),
        scratch_types=[
            pltpu.VMEM((EW_ROWS, 128), jnp.int32),
            pltpu.VMEM((EW_ROWS, 128), jnp.int32),
            pltpu.VMEM((EW_ROWS, 128), jnp.float32),
            pltpu.VMEM((EW_ROWS, 128), jnp.float32),
            pltpu.VMEM((N_PAD,), jnp.float32),
            pltpu.VMEM((ACC_PAD // 16,), jnp.float32),
            pltpu.VMEM_SHARED((ACC_PAD,), jnp.float32),
        ],
    )
    return kfn(idxi3, idxj3, len3, q_pad)


def _tc_recip_body(pos_ref, batch_ref, q_ref, base_ref, er_ref,
                   km_ref, sym_ref, r2v_ref, r2s_ref, vfac_ref,
                   out_ref, sfre_scr, sfim_scr, sfac_scr):
    p = pl.program_id(0)
    i = pl.program_id(1)

    @pl.when(jnp.logical_and(p == 0, i == 0))
    def _():
        sfre_scr[...] = jnp.zeros((B_STRUCT, K_PAD), jnp.float32)
        sfim_scr[...] = jnp.zeros((B_STRUCT, K_PAD), jnp.float32)
        km0 = km_ref[0:1, :]
        km1 = km_ref[1:2, :]
        km2 = km_ref[2:3, :]
        for b in range(B_STRUCT):
            kv0 = r2s_ref[b, 0] * km0 + r2s_ref[b, 3] * km1 + r2s_ref[b, 6] * km2
            kv1 = r2s_ref[b, 1] * km0 + r2s_ref[b, 4] * km1 + r2s_ref[b, 7] * km2
            kv2 = r2s_ref[b, 2] * km0 + r2s_ref[b, 5] * km1 + r2s_ref[b, 8] * km2
            ksq = kv0 * kv0 + kv1 * kv1 + kv2 * kv2
            kfac = jnp.where(ksq > 0.0,
                             jnp.exp(ksq * -0.25) / jnp.maximum(ksq, 1e-30),
                             jnp.zeros((1, K_PAD), jnp.float32))
            sfac_scr[b:b + 1, :] = vfac_ref[b] * sym_ref[...] * kfac

    pos = pos_ref[...]
    bidx = batch_ref[...]
    q = q_ref[...]
    onehot = (bidx == lax.broadcasted_iota(jnp.int32, (1, B_STRUCT), 1)
              ).astype(jnp.float32)
    rm = lax.dot_general(onehot, r2v_ref[...], (((1,), (0,)), ((), ())),
                         preferred_element_type=jnp.float32,
                         precision=lax.Precision.HIGHEST)
    px = pos[:, 0:1]
    py = pos[:, 1:2]
    pz = pos[:, 2:3]
    proj0 = px * rm[:, 0:1] + py * rm[:, 1:2] + pz * rm[:, 2:3]
    proj1 = px * rm[:, 3:4] + py * rm[:, 4:5] + pz * rm[:, 5:6]
    proj2 = px * rm[:, 6:7] + py * rm[:, 7:8] + pz * rm[:, 8:9]
    proj0 = proj0.astype(jnp.bfloat16).astype(jnp.float32)
    proj1 = proj1.astype(jnp.bfloat16).astype(jnp.float32)
    proj2 = proj2.astype(jnp.bfloat16).astype(jnp.float32)
    kr = (proj0 * km_ref[0:1, :] + proj1 * km_ref[1:2, :]
          + proj2 * km_ref[2:3, :])
    nred = jnp.round(kr * INV_TWO_PI)
    krr = ((kr - nred * TP_HI) - nred * TP_MID) - nred * TP_LO
    cos_kr = jnp.cos(krr)
    sin_kr = jnp.sin(krr)

    @pl.when(p == 0)
    def _():
        qoh = onehot * q
        sfre_scr[...] += lax.dot_general(
            qoh, cos_kr, (((0,), (0,)), ((), ())),
            preferred_element_type=jnp.float32,
            precision=lax.Precision.HIGHEST)
        sfim_scr[...] += lax.dot_general(
            qoh, sin_kr, (((0,), (0,)), ((), ())),
            preferred_element_type=jnp.float32,
            precision=lax.Precision.HIGHEST)
        out_ref[...] = jnp.zeros((BN, 1), jnp.float32)

    @pl.when(p == 1)
    def _():
        a_mat = sfac_scr[...] * sfre_scr[...]
        c_mat = sfac_scr[...] * sfim_scr[...]
        ag = lax.dot_general(onehot, a_mat, (((1,), (0,)), ((), ())),
                             preferred_element_type=jnp.float32,
                             precision=lax.Precision.HIGHEST)
        cg = lax.dot_general(onehot, c_mat, (((1,), (0,)), ((), ())),
                             preferred_element_type=jnp.float32,
                             precision=lax.Precision.HIGHEST)
        per_atom = jnp.sum(cos_kr * ag + sin_kr * cg, axis=1,
                           keepdims=True)
        er = er_ref[:, 0:1] + er_ref[:, 1:2]
        out_ref[...] = (base_ref[...] + er + q * per_atom
                        - SELF_COEF * q * q)


def _recip_tc(pos_p, batch_p, q_p, base_p, er_p, r2_flat, vfac):
    km_t = jnp.asarray(_KM_T)
    sym = jnp.asarray(_SYM_PAD)
    grid = (2, NB)
    return pl.pallas_call(
        _tc_recip_body,
        grid=grid,
        in_specs=[
            pl.BlockSpec((BN, 3), lambda p, i: (i, 0)),
            pl.BlockSpec((BN, 1), lambda p, i: (i, 0)),
            pl.BlockSpec((BN, 1), lambda p, i: (i, 0)),
            pl.BlockSpec((BN, 1), lambda p, i: (i, 0)),
            pl.BlockSpec((BN, 2), lambda p, i: (i, 0)),
            pl.BlockSpec((3, K_PAD), lambda p, i: (0, 0)),
            pl.BlockSpec((1, K_PAD), lambda p, i: (0, 0)),
            pl.BlockSpec((B_STRUCT, 9), lambda p, i: (0, 0)),
            pl.BlockSpec(memory_space=pltpu.SMEM),
            pl.BlockSpec(memory_space=pltpu.SMEM),
        ],
        out_specs=pl.BlockSpec((BN, 1), lambda p, i: (i, 0)),
        out_shape=jax.ShapeDtypeStruct((N_PAD, 1), jnp.float32),
        scratch_shapes=[
            pltpu.VMEM((B_STRUCT, K_PAD), jnp.float32),
            pltpu.VMEM((B_STRUCT, K_PAD), jnp.float32),
            pltpu.VMEM((B_STRUCT, K_PAD), jnp.float32),
        ],
    )(pos_p, batch_p, q_p, base_p, er_p, km_t, sym, r2_flat, r2_flat, vfac)


def _inv3x3_t(cell):
    a = cell
    c00 = a[:, 1, 1] * a[:, 2, 2] - a[:, 1, 2] * a[:, 2, 1]
    c01 = a[:, 1, 2] * a[:, 2, 0] - a[:, 1, 0] * a[:, 2, 2]
    c02 = a[:, 1, 0] * a[:, 2, 1] - a[:, 1, 1] * a[:, 2, 0]
    c10 = a[:, 0, 2] * a[:, 2, 1] - a[:, 0, 1] * a[:, 2, 2]
    c11 = a[:, 0, 0] * a[:, 2, 2] - a[:, 0, 2] * a[:, 2, 0]
    c12 = a[:, 0, 1] * a[:, 2, 0] - a[:, 0, 0] * a[:, 2, 1]
    c20 = a[:, 0, 1] * a[:, 1, 2] - a[:, 0, 2] * a[:, 1, 1]
    c21 = a[:, 0, 2] * a[:, 1, 0] - a[:, 0, 0] * a[:, 1, 2]
    c22 = a[:, 0, 0] * a[:, 1, 1] - a[:, 0, 1] * a[:, 1, 0]
    det = a[:, 0, 0] * c00 + a[:, 0, 1] * c01 + a[:, 0, 2] * c02
    cof = jnp.stack([
        jnp.stack([c00, c01, c02], axis=-1),
        jnp.stack([c10, c11, c12], axis=-1),
        jnp.stack([c20, c21, c22], axis=-1),
    ], axis=1)
    r2 = TWO_PI * cof / det[:, None, None]
    return r2, jnp.abs(det)


def kernel(positions, cell, lengths, partial_charges, atomic_energies,
           edge_index, batch, n_atoms):
    del n_atoms
    r2, absdet = _inv3x3_t(cell)
    r2_flat = r2.reshape(B_STRUCT, 9)
    vfac = KE * TWO_PI / absdet

    idxi = edge_index[0].astype(jnp.int32)
    idxj = edge_index[1].astype(jnp.int32)
    pad_e = E_PAD - E_EDGES
    idxi3 = jnp.pad(idxi, (0, pad_e)).reshape(NW, EW_ROWS, 128)
    idxj3 = jnp.pad(idxj, (0, pad_e)).reshape(NW, EW_ROWS, 128)
    len3 = jnp.pad(lengths, (0, pad_e), constant_values=1e9
                   ).reshape(NW, EW_ROWS, 128)
    q_pad = jnp.pad(partial_charges, (0, N_PAD - N_ATOMS))

    er = _real_space_sc(idxi3, idxj3, len3, q_pad)
    er_p = er.T

    pos_p = jnp.pad(positions, ((0, N_PAD - N_ATOMS), (0, 0)))
    batch_p = jnp.pad(batch.astype(jnp.int32), (0, N_PAD - N_ATOMS)
                      ).reshape(N_PAD, 1)
    q_p2 = q_pad.reshape(N_PAD, 1)
    base_p = jnp.pad(atomic_energies, (0, N_PAD - N_ATOMS)).reshape(N_PAD, 1)
    out = _recip_tc(pos_p, batch_p, q_p2, base_p, er_p, r2_flat, vfac)
    return out[:N_ATOMS, 0]


# speedup vs baseline: 24.7432x; 1.6305x over previous
import functools

import numpy as np
import jax
import jax.numpy as jnp
from jax import lax
from jax.experimental import pallas as pl
from jax.experimental.pallas import tpu as pltpu
from jax.experimental.pallas import tpu_sc as plsc

KE = 14.399645351950548
KE_HALF = KE / 2.0
ALPHA = 1.0
R_CUTOFF = 5.0
K_MAX = 10
TWO_PI = 2.0 * np.pi
SQRT_PI = float(np.sqrt(np.pi))
SELF_COEF = KE * ALPHA / SQRT_PI

AS_P = 0.3275911
AS_A1 = 0.254829592
AS_A2 = -0.284496736
AS_A3 = 1.421413741
AS_A4 = -1.453152027
AS_A5 = 1.061405429

INV_TWO_PI = 0.15915494309189535
TP_HI = 6.283203125
TP_MID = -1.7818063497543335e-05
TP_LO = 2.430837753308879e-10

N_ATOMS = 10000
N_PAD = 10240
BN = 512
NB = N_PAD // BN
B_STRUCT = 16
E_EDGES = 640000
NW = 32
EW_ROWS = 157
EW = EW_ROWS * 128
E_PAD = NW * EW
ACC_PAD = N_PAD


def _kmesh_half():
    k_idxs = np.arange(-K_MAX, K_MAX + 1).astype(np.float32)
    ku, kv, kw = np.meshgrid(k_idxs[K_MAX:], k_idxs, k_idxs, indexing='ij')
    km = np.stack([ku, kv, kw], axis=-1).reshape(-1, 3)
    ksq = (km ** 2).sum(-1)
    mask = (ksq > 0) & (ksq <= K_MAX ** 2)
    km = km[mask]
    sym = np.where(km[:, 0] == 0.0, 1.0, 2.0).astype(np.float32)
    return km.astype(np.float32), sym


_KM, _SYM = _kmesh_half()
K_REAL = _KM.shape[0]
K_PAD = ((K_REAL + 127) // 128) * 128
_KM_T = np.zeros((3, K_PAD), np.float32)
_KM_T[:, :K_REAL] = _KM.T
_SYM_PAD = np.zeros((1, K_PAD), np.float32)
_SYM_PAD[0, :K_REAL] = _SYM


def _sc_real_body(idxi_hbm, idxj_hbm, len_hbm, q_hbm, out_hbm,
                  idxi_v, idxj_v, len_v, vals_v, q_v, z_v, acc_sh):
    c = lax.axis_index("c")
    s = lax.axis_index("s")
    wid = c * 16 + s

    def zbody(i, carry):
        z_v[pl.ds(i * 16, 16)] = jnp.zeros((16,), jnp.float32)
        return carry
    lax.fori_loop(0, (ACC_PAD // 16) // 16, zbody, 0)
    pltpu.sync_copy(z_v, acc_sh.at[pl.ds(s * (ACC_PAD // 16), ACC_PAD // 16)])

    pltpu.sync_copy(idxi_hbm.at[wid], idxi_v)
    pltpu.sync_copy(idxj_hbm.at[wid], idxj_v)
    pltpu.sync_copy(len_hbm.at[wid], len_v)
    pltpu.sync_copy(q_hbm, q_v)

    def cbody(t, carry):
        j = t // 8
        col = (t % 8) * 16
        ii = idxi_v[j, pl.ds(col, 16)]
        jj = idxj_v[j, pl.ds(col, 16)]
        ll = len_v[j, pl.ds(col, 16)]
        qi = plsc.load_gather(q_v, [ii])
        qj = plsc.load_gather(q_v, [jj])
        tt = 1.0 / (1.0 + AS_P * ll)
        poly = tt * (AS_A1 + tt * (AS_A2 + tt * (AS_A3 + tt * (AS_A4 + tt * AS_A5))))
        erfc_v = poly * jnp.exp(-ll * ll)
        pv = KE_HALF * qi * qj * erfc_v / ll
        pv = jnp.where(ll <= R_CUTOFF, pv, jnp.zeros((16,), jnp.float32))
        vals_v[j, pl.ds(col, 16)] = pv
        return carry
    lax.fori_loop(0, EW_ROWS * 8, cbody, 0)

    plsc.subcore_barrier()

    def sbody(j, carry):
        pltpu.sync_copy(vals_v.at[j], acc_sh.at[idxi_v.at[j]], add=True)
        return carry
    lax.fori_loop(0, EW_ROWS, sbody, 0)

    plsc.subcore_barrier()

    @pl.when(s == 0)
    def _():
        pltpu.sync_copy(acc_sh, out_hbm.at[c])


def _real_space_sc(idxi3, idxj3, len3, q_pad):
    mesh = plsc.VectorSubcoreMesh(core_axis_name="c", subcore_axis_name="s")
    kfn = pl.kernel(
        _sc_real_body,
        mesh=mesh,
        compiler_params=pltpu.CompilerParams(needs_layout_passes=False),
        out_type=jax.ShapeDtypeStruct((2, ACC_PAD), jnp.float32),
        scratch_types=[
            pltpu.VMEM((EW_ROWS, 128), jnp.int32),
            pltpu.VMEM((EW_ROWS, 128), jnp.int32),
            pltpu.VMEM((EW_ROWS, 128), jnp.float32),
            pltpu.VMEM((EW_ROWS, 128), jnp.float32),
            pltpu.VMEM((N_PAD,), jnp.float32),
            pltpu.VMEM((ACC_PAD // 16,), jnp.float32),
            pltpu.VMEM_SHARED((ACC_PAD,), jnp.float32),
        ],
    )
    return kfn(idxi3, idxj3, len3, q_pad)


def _tc_passa_body(pos_ref, batch_ref, q_ref, km_ref, r2v_ref,
                   cos_ref, sin_ref, sf_ref, sfre_scr, sfim_scr):
    i = pl.program_id(0)

    @pl.when(i == 0)
    def _():
        sfre_scr[...] = jnp.zeros((B_STRUCT, K_PAD), jnp.float32)
        sfim_scr[...] = jnp.zeros((B_STRUCT, K_PAD), jnp.float32)

    pos = pos_ref[...]
    bidx = batch_ref[...]
    q = q_ref[...]
    onehot = (bidx == lax.broadcasted_iota(jnp.int32, (1, B_STRUCT), 1)
              ).astype(jnp.float32)
    rm = lax.dot_general(onehot, r2v_ref[...], (((1,), (0,)), ((), ())),
                         preferred_element_type=jnp.float32,
                         precision=lax.Precision.HIGHEST)
    px = pos[:, 0:1]
    py = pos[:, 1:2]
    pz = pos[:, 2:3]
    proj0 = px * rm[:, 0:1] + py * rm[:, 1:2] + pz * rm[:, 2:3]
    proj1 = px * rm[:, 3:4] + py * rm[:, 4:5] + pz * rm[:, 5:6]
    proj2 = px * rm[:, 6:7] + py * rm[:, 7:8] + pz * rm[:, 8:9]
    proj0 = proj0.astype(jnp.bfloat16).astype(jnp.float32)
    proj1 = proj1.astype(jnp.bfloat16).astype(jnp.float32)
    proj2 = proj2.astype(jnp.bfloat16).astype(jnp.float32)
    kr = (proj0 * km_ref[0:1, :] + proj1 * km_ref[1:2, :]
          + proj2 * km_ref[2:3, :])
    nred = jnp.round(kr * INV_TWO_PI)
    krr = ((kr - nred * TP_HI) - nred * TP_MID) - nred * TP_LO
    cos_kr = jnp.cos(krr)
    sin_kr = jnp.sin(krr)
    cos_ref[...] = cos_kr
    sin_ref[...] = sin_kr

    qoh = onehot * q
    sfre_scr[...] += lax.dot_general(
        qoh, cos_kr, (((0,), (0,)), ((), ())),
        preferred_element_type=jnp.float32,
        precision=lax.Precision.HIGHEST)
    sfim_scr[...] += lax.dot_general(
        qoh, sin_kr, (((0,), (0,)), ((), ())),
        preferred_element_type=jnp.float32,
        precision=lax.Precision.HIGHEST)

    @pl.when(i == NB - 1)
    def _():
        sf_ref[0, :, :] = sfre_scr[...]
        sf_ref[1, :, :] = sfim_scr[...]


def _tc_passb_body(cos_ref, sin_ref, sf_ref, batch_ref, q_ref, base_ref,
                   er_ref, km_ref, sym_ref, r2s_ref, vfac_ref,
                   out_ref, ac_scr):
    i = pl.program_id(0)

    @pl.when(i == 0)
    def _():
        km0 = km_ref[0:1, :]
        km1 = km_ref[1:2, :]
        km2 = km_ref[2:3, :]
        for b in range(B_STRUCT):
            kv0 = r2s_ref[b, 0] * km0 + r2s_ref[b, 3] * km1 + r2s_ref[b, 6] * km2
            kv1 = r2s_ref[b, 1] * km0 + r2s_ref[b, 4] * km1 + r2s_ref[b, 7] * km2
            kv2 = r2s_ref[b, 2] * km0 + r2s_ref[b, 5] * km1 + r2s_ref[b, 8] * km2
            ksq = kv0 * kv0 + kv1 * kv1 + kv2 * kv2
            kfac = jnp.where(ksq > 0.0,
                             jnp.exp(ksq * -0.25) / jnp.maximum(ksq, 1e-30),
                             jnp.zeros((1, K_PAD), jnp.float32))
            sfac_row = vfac_ref[b] * sym_ref[...] * kfac
            ac_scr[b:b + 1, :] = sfac_row * sf_ref[0, b:b + 1, :]
            ac_scr[B_STRUCT + b:B_STRUCT + b + 1, :] = \
                sfac_row * sf_ref[1, b:b + 1, :]

    bidx = batch_ref[...]
    q = q_ref[...]
    onehot = (bidx == lax.broadcasted_iota(jnp.int32, (1, B_STRUCT), 1)
              ).astype(jnp.float32)
    t_re = lax.dot_general(cos_ref[...], ac_scr[0:B_STRUCT, :],
                           (((1,), (1,)), ((), ())),
                           preferred_element_type=jnp.float32,
                           precision=lax.Precision.HIGHEST)
    t_im = lax.dot_general(sin_ref[...], ac_scr[B_STRUCT:2 * B_STRUCT, :],
                           (((1,), (1,)), ((), ())),
                           preferred_element_type=jnp.float32,
                           precision=lax.Precision.HIGHEST)
    per_atom = jnp.sum(onehot * (t_re + t_im), axis=1, keepdims=True)
    er = er_ref[:, 0:1] + er_ref[:, 1:2]
    out_ref[...] = (base_ref[...] + er + q * per_atom - SELF_COEF * q * q)


def _recip_tc(pos_p, batch_p, q_p, base_p, er_p, r2_flat, vfac):
    km_t = jnp.asarray(_KM_T)
    sym = jnp.asarray(_SYM_PAD)
    cos_a, sin_a, sf = pl.pallas_call(
        _tc_passa_body,
        grid=(NB,),
        in_specs=[
            pl.BlockSpec((BN, 3), lambda i: (i, 0)),
            pl.BlockSpec((BN, 1), lambda i: (i, 0)),
            pl.BlockSpec((BN, 1), lambda i: (i, 0)),
            pl.BlockSpec((3, K_PAD), lambda i: (0, 0)),
            pl.BlockSpec((B_STRUCT, 9), lambda i: (0, 0)),
        ],
        out_specs=[
            pl.BlockSpec((BN, K_PAD), lambda i: (i, 0)),
            pl.BlockSpec((BN, K_PAD), lambda i: (i, 0)),
            pl.BlockSpec((2, B_STRUCT, K_PAD), lambda i: (0, 0, 0)),
        ],
        out_shape=[
            jax.ShapeDtypeStruct((N_PAD, K_PAD), jnp.float32),
            jax.ShapeDtypeStruct((N_PAD, K_PAD), jnp.float32),
            jax.ShapeDtypeStruct((2, B_STRUCT, K_PAD), jnp.float32),
        ],
        scratch_shapes=[
            pltpu.VMEM((B_STRUCT, K_PAD), jnp.float32),
            pltpu.VMEM((B_STRUCT, K_PAD), jnp.float32),
        ],
    )(pos_p, batch_p, q_p, km_t, r2_flat)

    return pl.pallas_call(
        _tc_passb_body,
        grid=(NB,),
        in_specs=[
            pl.BlockSpec((BN, K_PAD), lambda i: (i, 0)),
            pl.BlockSpec((BN, K_PAD), lambda i: (i, 0)),
            pl.BlockSpec((2, B_STRUCT, K_PAD), lambda i: (0, 0, 0)),
            pl.BlockSpec((BN, 1), lambda i: (i, 0)),
            pl.BlockSpec((BN, 1), lambda i: (i, 0)),
            pl.BlockSpec((BN, 1), lambda i: (i, 0)),
            pl.BlockSpec((BN, 2), lambda i: (i, 0)),
            pl.BlockSpec((3, K_PAD), lambda i: (0, 0)),
            pl.BlockSpec((1, K_PAD), lambda i: (0, 0)),
            pl.BlockSpec(memory_space=pltpu.SMEM),
            pl.BlockSpec(memory_space=pltpu.SMEM),
        ],
        out_specs=pl.BlockSpec((BN, 1), lambda i: (i, 0)),
        out_shape=jax.ShapeDtypeStruct((N_PAD, 1), jnp.float32),
        scratch_shapes=[
            pltpu.VMEM((2 * B_STRUCT, K_PAD), jnp.float32),
        ],
    )(cos_a, sin_a, sf, batch_p, q_p, base_p, er_p, km_t, sym,
      r2_flat, vfac)


def _inv3x3_t(cell):
    a = cell
    c00 = a[:, 1, 1] * a[:, 2, 2] - a[:, 1, 2] * a[:, 2, 1]
    c01 = a[:, 1, 2] * a[:, 2, 0] - a[:, 1, 0] * a[:, 2, 2]
    c02 = a[:, 1, 0] * a[:, 2, 1] - a[:, 1, 1] * a[:, 2, 0]
    c10 = a[:, 0, 2] * a[:, 2, 1] - a[:, 0, 1] * a[:, 2, 2]
    c11 = a[:, 0, 0] * a[:, 2, 2] - a[:, 0, 2] * a[:, 2, 0]
    c12 = a[:, 0, 1] * a[:, 2, 0] - a[:, 0, 0] * a[:, 2, 1]
    c20 = a[:, 0, 1] * a[:, 1, 2] - a[:, 0, 2] * a[:, 1, 1]
    c21 = a[:, 0, 2] * a[:, 1, 0] - a[:, 0, 0] * a[:, 1, 2]
    c22 = a[:, 0, 0] * a[:, 1, 1] - a[:, 0, 1] * a[:, 1, 0]
    det = a[:, 0, 0] * c00 + a[:, 0, 1] * c01 + a[:, 0, 2] * c02
    cof = jnp.stack([
        jnp.stack([c00, c01, c02], axis=-1),
        jnp.stack([c10, c11, c12], axis=-1),
        jnp.stack([c20, c21, c22], axis=-1),
    ], axis=1)
    r2 = TWO_PI * cof / det[:, None, None]
    return r2, jnp.abs(det)


def kernel(positions, cell, lengths, partial_charges, atomic_energies,
           edge_index, batch, n_atoms):
    del n_atoms
    r2, absdet = _inv3x3_t(cell)
    r2_flat = r2.reshape(B_STRUCT, 9)
    vfac = KE * TWO_PI / absdet

    idxi = edge_index[0].astype(jnp.int32)
    idxj = edge_index[1].astype(jnp.int32)
    pad_e = E_PAD - E_EDGES
    idxi3 = jnp.pad(idxi, (0, pad_e)).reshape(NW, EW_ROWS, 128)
    idxj3 = jnp.pad(idxj, (0, pad_e)).reshape(NW, EW_ROWS, 128)
    len3 = jnp.pad(lengths, (0, pad_e), constant_values=1e9
                   ).reshape(NW, EW_ROWS, 128)
    q_pad = jnp.pad(partial_charges, (0, N_PAD - N_ATOMS))

    er = _real_space_sc(idxi3, idxj3, len3, q_pad)
    er_p = er.T

    pos_p = jnp.pad(positions, ((0, N_PAD - N_ATOMS), (0, 0)))
    batch_p = jnp.pad(batch.astype(jnp.int32), (0, N_PAD - N_ATOMS)
                      ).reshape(N_PAD, 1)
    q_p2 = q_pad.reshape(N_PAD, 1)
    base_p = jnp.pad(atomic_energies, (0, N_PAD - N_ATOMS)).reshape(N_PAD, 1)
    out = _recip_tc(pos_p, batch_p, q_p2, base_p, er_p, r2_flat, vfac)
    return out[:N_ATOMS, 0]


# speedup vs baseline: 25.1342x; 1.0158x over previous
import functools

import numpy as np
import jax
import jax.numpy as jnp
from jax import lax
from jax.experimental import pallas as pl
from jax.experimental.pallas import tpu as pltpu
from jax.experimental.pallas import tpu_sc as plsc

KE = 14.399645351950548
KE_HALF = KE / 2.0
ALPHA = 1.0
R_CUTOFF = 5.0
K_MAX = 10
TWO_PI = 2.0 * np.pi
SQRT_PI = float(np.sqrt(np.pi))
SELF_COEF = KE * ALPHA / SQRT_PI

AS_P = 0.3275911
AS_A1 = 0.254829592
AS_A2 = -0.284496736
AS_A3 = 1.421413741
AS_A4 = -1.453152027
AS_A5 = 1.061405429

INV_TWO_PI = 0.15915494309189535
TP_HI = 6.283203125
TP_MID = -1.7818063497543335e-05
TP_LO = 2.430837753308879e-10

N_ATOMS = 10000
N_PAD = 10240
BN = 512
NB = N_PAD // BN
B_STRUCT = 16
E_EDGES = 640000
NW = 32
EW_ROWS = 157
EW = EW_ROWS * 128
E_PAD = NW * EW
ACC_PAD = N_PAD


def _kmesh_half():
    k_idxs = np.arange(-K_MAX, K_MAX + 1).astype(np.float32)
    ku, kv, kw = np.meshgrid(k_idxs[K_MAX:], k_idxs, k_idxs, indexing='ij')
    km = np.stack([ku, kv, kw], axis=-1).reshape(-1, 3)
    ksq = (km ** 2).sum(-1)
    mask = (ksq > 0) & (ksq <= K_MAX ** 2)
    km = km[mask]
    sym = np.where(km[:, 0] == 0.0, 1.0, 2.0).astype(np.float32)
    return km.astype(np.float32), sym


_KM, _SYM = _kmesh_half()
K_REAL = _KM.shape[0]
K_PAD = ((K_REAL + 127) // 128) * 128
_KM_T = np.zeros((3, K_PAD), np.float32)
_KM_T[:, :K_REAL] = _KM.T
_SYM_PAD = np.zeros((1, K_PAD), np.float32)
_SYM_PAD[0, :K_REAL] = _SYM


def _sc_real_body(idxi_hbm, idxj_hbm, len_hbm, q_hbm, out_hbm,
                  idxi_v, idxj_v, len_v, vals_v, q_v, z_v, acc_sh):
    c = lax.axis_index("c")
    s = lax.axis_index("s")
    wid = c * 16 + s

    def zbody(i, carry):
        z_v[pl.ds(i * 16, 16)] = jnp.zeros((16,), jnp.float32)
        return carry
    lax.fori_loop(0, (ACC_PAD // 16) // 16, zbody, 0)
    pltpu.sync_copy(z_v, acc_sh.at[pl.ds(s * (ACC_PAD // 16), ACC_PAD // 16)])

    pltpu.sync_copy(idxi_hbm.at[wid], idxi_v)
    pltpu.sync_copy(idxj_hbm.at[wid], idxj_v)
    pltpu.sync_copy(len_hbm.at[wid], len_v)
    pltpu.sync_copy(q_hbm, q_v)

    def cbody(t, carry):
        j = t // 8
        col = (t % 8) * 16
        ii = idxi_v[j, pl.ds(col, 16)]
        jj = idxj_v[j, pl.ds(col, 16)]
        ll = len_v[j, pl.ds(col, 16)]
        qi = plsc.load_gather(q_v, [ii])
        qj = plsc.load_gather(q_v, [jj])
        tt = 1.0 / (1.0 + AS_P * ll)
        poly = tt * (AS_A1 + tt * (AS_A2 + tt * (AS_A3 + tt * (AS_A4 + tt * AS_A5))))
        erfc_v = poly * jnp.exp(-ll * ll)
        pv = KE_HALF * qi * qj * erfc_v / ll
        pv = jnp.where(ll <= R_CUTOFF, pv, jnp.zeros((16,), jnp.float32))
        vals_v[j, pl.ds(col, 16)] = pv
        return carry
    lax.fori_loop(0, EW_ROWS * 8, cbody, 0)

    plsc.subcore_barrier()

    def sbody(j, carry):
        pltpu.sync_copy(vals_v.at[j], acc_sh.at[idxi_v.at[j]], add=True)
        return carry
    lax.fori_loop(0, EW_ROWS, sbody, 0)

    plsc.subcore_barrier()

    @pl.when(s == 0)
    def _():
        pltpu.sync_copy(acc_sh, out_hbm.at[c])


def _real_space_sc(idxi3, idxj3, len3, q_pad):
    mesh = plsc.VectorSubcoreMesh(core_axis_name="c", subcore_axis_name="s")
    kfn = pl.kernel(
        _sc_real_body,
        mesh=mesh,
        compiler_params=pltpu.CompilerParams(needs_layout_passes=False),
        out_type=jax.ShapeDtypeStruct((2, ACC_PAD), jnp.float32),
        scratch_types=[
            pltpu.VMEM((EW_ROWS, 128), jnp.int32),
            pltpu.VMEM((EW_ROWS, 128), jnp.int32),
            pltpu.VMEM((EW_ROWS, 128), jnp.float32),
            pltpu.VMEM((EW_ROWS, 128), jnp.float32),
            pltpu.VMEM((N_PAD,), jnp.float32),
            pltpu.VMEM((ACC_PAD // 16,), jnp.float32),
            pltpu.VMEM_SHARED((ACC_PAD,), jnp.float32),
        ],
    )
    return kfn(idxi3, idxj3, len3, q_pad)


def _tc_passa_body(pos_ref, batch_ref, q_ref, km_ref, r2v_ref,
                   cos_ref, sin_ref, sf_ref, sfre_scr, sfim_scr):
    i = pl.program_id(0)

    @pl.when(i == 0)
    def _():
        sfre_scr[...] = jnp.zeros((B_STRUCT, K_PAD), jnp.float32)
        sfim_scr[...] = jnp.zeros((B_STRUCT, K_PAD), jnp.float32)

    pos = pos_ref[...]
    bidx = batch_ref[...]
    q = q_ref[...]
    onehot = (bidx == lax.broadcasted_iota(jnp.int32, (1, B_STRUCT), 1)
              ).astype(jnp.float32)
    rm = lax.dot_general(onehot, r2v_ref[...], (((1,), (0,)), ((), ())),
                         preferred_element_type=jnp.float32,
                         precision=lax.Precision.HIGHEST)
    px = pos[:, 0:1]
    py = pos[:, 1:2]
    pz = pos[:, 2:3]
    proj0 = px * rm[:, 0:1] + py * rm[:, 1:2] + pz * rm[:, 2:3]
    proj1 = px * rm[:, 3:4] + py * rm[:, 4:5] + pz * rm[:, 5:6]
    proj2 = px * rm[:, 6:7] + py * rm[:, 7:8] + pz * rm[:, 8:9]
    proj0 = proj0.astype(jnp.bfloat16).astype(jnp.float32)
    proj1 = proj1.astype(jnp.bfloat16).astype(jnp.float32)
    proj2 = proj2.astype(jnp.bfloat16).astype(jnp.float32)
    kr = (proj0 * km_ref[0:1, :] + proj1 * km_ref[1:2, :]
          + proj2 * km_ref[2:3, :])
    nred = jnp.round(kr * INV_TWO_PI)
    krr = ((kr - nred * TP_HI) - nred * TP_MID) - nred * TP_LO
    cos_kr = jnp.cos(krr)
    sin_kr = jnp.sin(krr)
    cos_ref[...] = cos_kr.astype(jnp.bfloat16)
    sin_ref[...] = sin_kr.astype(jnp.bfloat16)

    qoh = onehot * q
    sfre_scr[...] += lax.dot_general(
        qoh, cos_kr, (((0,), (0,)), ((), ())),
        preferred_element_type=jnp.float32,
        precision=lax.Precision.HIGHEST)
    sfim_scr[...] += lax.dot_general(
        qoh, sin_kr, (((0,), (0,)), ((), ())),
        preferred_element_type=jnp.float32,
        precision=lax.Precision.HIGHEST)

    @pl.when(i == NB - 1)
    def _():
        sf_ref[0, :, :] = sfre_scr[...]
        sf_ref[1, :, :] = sfim_scr[...]


def _tc_passb_body(cos_ref, sin_ref, sf_ref, batch_ref, q_ref, base_ref,
                   er_ref, km_ref, sym_ref, r2s_ref, vfac_ref,
                   out_ref, ac_scr):
    i = pl.program_id(0)

    @pl.when(i == 0)
    def _():
        km0 = km_ref[0:1, :]
        km1 = km_ref[1:2, :]
        km2 = km_ref[2:3, :]
        for b in range(B_STRUCT):
            kv0 = r2s_ref[b, 0] * km0 + r2s_ref[b, 3] * km1 + r2s_ref[b, 6] * km2
            kv1 = r2s_ref[b, 1] * km0 + r2s_ref[b, 4] * km1 + r2s_ref[b, 7] * km2
            kv2 = r2s_ref[b, 2] * km0 + r2s_ref[b, 5] * km1 + r2s_ref[b, 8] * km2
            ksq = kv0 * kv0 + kv1 * kv1 + kv2 * kv2
            kfac = jnp.where(ksq > 0.0,
                             jnp.exp(ksq * -0.25) / jnp.maximum(ksq, 1e-30),
                             jnp.zeros((1, K_PAD), jnp.float32))
            sfac_row = vfac_ref[b] * sym_ref[...] * kfac
            ac_scr[b:b + 1, :] = sfac_row * sf_ref[0, b:b + 1, :]
            ac_scr[B_STRUCT + b:B_STRUCT + b + 1, :] = \
                sfac_row * sf_ref[1, b:b + 1, :]

    bidx = batch_ref[...]
    q = q_ref[...]
    onehot = (bidx == lax.broadcasted_iota(jnp.int32, (1, B_STRUCT), 1)
              ).astype(jnp.float32)
    t_re = lax.dot_general(cos_ref[...], ac_scr[0:B_STRUCT, :].astype(jnp.bfloat16),
                           (((1,), (1,)), ((), ())),
                           preferred_element_type=jnp.float32)
    t_im = lax.dot_general(sin_ref[...], ac_scr[B_STRUCT:2 * B_STRUCT, :].astype(jnp.bfloat16),
                           (((1,), (1,)), ((), ())),
                           preferred_element_type=jnp.float32)
    per_atom = jnp.sum(onehot * (t_re + t_im), axis=1, keepdims=True)
    er = er_ref[:, 0:1] + er_ref[:, 1:2]
    out_ref[...] = (base_ref[...] + er + q * per_atom - SELF_COEF * q * q)


def _recip_tc(pos_p, batch_p, q_p, base_p, er_p, r2_flat, vfac):
    km_t = jnp.asarray(_KM_T)
    sym = jnp.asarray(_SYM_PAD)
    cos_a, sin_a, sf = pl.pallas_call(
        _tc_passa_body,
        grid=(NB,),
        in_specs=[
            pl.BlockSpec((BN, 3), lambda i: (i, 0)),
            pl.BlockSpec((BN, 1), lambda i: (i, 0)),
            pl.BlockSpec((BN, 1), lambda i: (i, 0)),
            pl.BlockSpec((3, K_PAD), lambda i: (0, 0)),
            pl.BlockSpec((B_STRUCT, 9), lambda i: (0, 0)),
        ],
        out_specs=[
            pl.BlockSpec((BN, K_PAD), lambda i: (i, 0)),
            pl.BlockSpec((BN, K_PAD), lambda i: (i, 0)),
            pl.BlockSpec((2, B_STRUCT, K_PAD), lambda i: (0, 0, 0)),
        ],
        out_shape=[
            jax.ShapeDtypeStruct((N_PAD, K_PAD), jnp.bfloat16),
            jax.ShapeDtypeStruct((N_PAD, K_PAD), jnp.bfloat16),
            jax.ShapeDtypeStruct((2, B_STRUCT, K_PAD), jnp.float32),
        ],
        scratch_shapes=[
            pltpu.VMEM((B_STRUCT, K_PAD), jnp.float32),
            pltpu.VMEM((B_STRUCT, K_PAD), jnp.float32),
        ],
    )(pos_p, batch_p, q_p, km_t, r2_flat)

    return pl.pallas_call(
        _tc_passb_body,
        grid=(NB,),
        in_specs=[
            pl.BlockSpec((BN, K_PAD), lambda i: (i, 0)),
            pl.BlockSpec((BN, K_PAD), lambda i: (i, 0)),
            pl.BlockSpec((2, B_STRUCT, K_PAD), lambda i: (0, 0, 0)),
            pl.BlockSpec((BN, 1), lambda i: (i, 0)),
            pl.BlockSpec((BN, 1), lambda i: (i, 0)),
            pl.BlockSpec((BN, 1), lambda i: (i, 0)),
            pl.BlockSpec((BN, 2), lambda i: (i, 0)),
            pl.BlockSpec((3, K_PAD), lambda i: (0, 0)),
            pl.BlockSpec((1, K_PAD), lambda i: (0, 0)),
            pl.BlockSpec(memory_space=pltpu.SMEM),
            pl.BlockSpec(memory_space=pltpu.SMEM),
        ],
        out_specs=pl.BlockSpec((BN, 1), lambda i: (i, 0)),
        out_shape=jax.ShapeDtypeStruct((N_PAD, 1), jnp.float32),
        scratch_shapes=[
            pltpu.VMEM((2 * B_STRUCT, K_PAD), jnp.float32),
        ],
    )(cos_a, sin_a, sf, batch_p, q_p, base_p, er_p, km_t, sym,
      r2_flat, vfac)


def _inv3x3_t(cell):
    a = cell
    c00 = a[:, 1, 1] * a[:, 2, 2] - a[:, 1, 2] * a[:, 2, 1]
    c01 = a[:, 1, 2] * a[:, 2, 0] - a[:, 1, 0] * a[:, 2, 2]
    c02 = a[:, 1, 0] * a[:, 2, 1] - a[:, 1, 1] * a[:, 2, 0]
    c10 = a[:, 0, 2] * a[:, 2, 1] - a[:, 0, 1] * a[:, 2, 2]
    c11 = a[:, 0, 0] * a[:, 2, 2] - a[:, 0, 2] * a[:, 2, 0]
    c12 = a[:, 0, 1] * a[:, 2, 0] - a[:, 0, 0] * a[:, 2, 1]
    c20 = a[:, 0, 1] * a[:, 1, 2] - a[:, 0, 2] * a[:, 1, 1]
    c21 = a[:, 0, 2] * a[:, 1, 0] - a[:, 0, 0] * a[:, 1, 2]
    c22 = a[:, 0, 0] * a[:, 1, 1] - a[:, 0, 1] * a[:, 1, 0]
    det = a[:, 0, 0] * c00 + a[:, 0, 1] * c01 + a[:, 0, 2] * c02
    cof = jnp.stack([
        jnp.stack([c00, c01, c02], axis=-1),
        jnp.stack([c10, c11, c12], axis=-1),
        jnp.stack([c20, c21, c22], axis=-1),
    ], axis=1)
    r2 = TWO_PI * cof / det[:, None, None]
    return r2, jnp.abs(det)


def kernel(positions, cell, lengths, partial_charges, atomic_energies,
           edge_index, batch, n_atoms):
    del n_atoms
    r2, absdet = _inv3x3_t(cell)
    r2_flat = r2.reshape(B_STRUCT, 9)
    vfac = KE * TWO_PI / absdet

    idxi = edge_index[0].astype(jnp.int32)
    idxj = edge_index[1].astype(jnp.int32)
    pad_e = E_PAD - E_EDGES
    idxi3 = jnp.pad(idxi, (0, pad_e)).reshape(NW, EW_ROWS, 128)
    idxj3 = jnp.pad(idxj, (0, pad_e)).reshape(NW, EW_ROWS, 128)
    len3 = jnp.pad(lengths, (0, pad_e), constant_values=1e9
                   ).reshape(NW, EW_ROWS, 128)
    q_pad = jnp.pad(partial_charges, (0, N_PAD - N_ATOMS))

    er = _real_space_sc(idxi3, idxj3, len3, q_pad)
    er_p = er.T

    pos_p = jnp.pad(positions, ((0, N_PAD - N_ATOMS), (0, 0)))
    batch_p = jnp.pad(batch.astype(jnp.int32), (0, N_PAD - N_ATOMS)
                      ).reshape(N_PAD, 1)
    q_p2 = q_pad.reshape(N_PAD, 1)
    base_p = jnp.pad(atomic_energies, (0, N_PAD - N_ATOMS)).reshape(N_PAD, 1)
    out = _recip_tc(pos_p, batch_p, q_p2, base_p, er_p, r2_flat, vfac)
    return out[:N_ATOMS, 0]
